# baseline (speedup 1.0000x reference)
import functools

import jax
import jax.numpy as jnp
from jax import lax
from jax.experimental import pallas as pl
from jax.experimental.pallas import tpu as pltpu
from jax.experimental.pallas import tpu_sc as plsc

N = 1_000_000
NW = 32
PER_W = 31_248
MAIN = NW * PER_W
TAIL = N - MAIN
KCH = 7
CHUNK = PER_W // KCH
IN_IT = CHUNK // 16
NBINS = 64
ROW = 256
LN2 = 0.69314718055994530942
SQRT2 = 1.4142135623730951


def _vlog(x):
    bits = plsc.bitcast(x, jnp.int32)
    e = (bits >> 23) - 127
    m = plsc.bitcast((bits & 0x007FFFFF) | 0x3F800000, jnp.float32)
    big = m > SQRT2
    m = jnp.where(big, m * 0.5, m)
    ef = (e + big.astype(jnp.int32)).astype(jnp.float32)
    s = (m - 1.0) / (m + 1.0)
    t = s * s
    poly = 1.0 + t * (1.0 / 3.0 + t * (1.0 / 5.0 + t * (1.0 / 7.0 + t * (1.0 / 9.0))))
    return ef * LN2 + 2.0 * s * poly


def _sc_body(conf_hbm, logf_hbm, tgt_hbm, bnd_hbm, out_hbm,
             cbuf, lbuf, tbuf, ebuf_c, ebuf_l, ebuf_t, bnd, hc, hs, hr, acc, prow):
    nc = 2
    wid = lax.axis_index("s") * nc + lax.axis_index("c")
    base = wid * PER_W
    lane = lax.iota(jnp.int32, 16)
    zero16 = jnp.zeros((16,), jnp.float32)

    pltpu.sync_copy(bnd_hbm, bnd)
    for i in range(NBINS):
        hc[pl.ds(i * 16, 16)] = zero16
        hs[pl.ds(i * 16, 16)] = zero16
        hr[pl.ds(i * 16, 16)] = zero16
    acc[...] = zero16

    def sample16(i, cb, lb, tb):
        conf = cb[pl.ds(i * 16, 16)]
        tgt = tb[pl.ds(i * 16, 16)]
        i0 = lane * 3 + i * 48
        l0 = plsc.load_gather(lb, [i0])
        l1 = plsc.load_gather(lb, [i0 + 1])
        l2 = plsc.load_gather(lb, [i0 + 2])
        pred = jnp.where(l2 > jnp.maximum(l0, l1), 2, jnp.where(l1 > l0, 1, 0))
        corr = (pred == tgt).astype(jnp.float32)
        p = jnp.maximum(conf, 1e-12)
        q = jnp.where(corr > 0.5, p, 1.0 - p)
        acc[...] = acc[...] - _vlog(q)
        y = conf * 50.0
        iy = y.astype(jnp.int32)
        j = iy + (y > iy.astype(jnp.float32)).astype(jnp.int32) - 1
        jc = jnp.clip(j, 0, 49)
        lo = plsc.load_gather(bnd, [jc])
        hi = plsc.load_gather(bnd, [jc + 1])
        j2 = jc + (conf > hi).astype(jnp.int32) - (conf <= lo).astype(jnp.int32)
        valid = j2 >= 0
        hidx = lane * NBINS + jnp.maximum(j2, 0)
        plsc.addupdate_scatter(hc, [hidx], jnp.ones((16,), jnp.float32), mask=valid)
        plsc.addupdate_scatter(hs, [hidx], conf, mask=valid)
        plsc.addupdate_scatter(hr, [hidx], corr, mask=valid)

    def chunk_body(g, _):
        off = base + g * CHUNK
        pltpu.sync_copy(conf_hbm.at[pl.ds(off, CHUNK)], cbuf)
        pltpu.sync_copy(logf_hbm.at[pl.ds(off * 3, CHUNK * 3)], lbuf)
        pltpu.sync_copy(tgt_hbm.at[pl.ds(off, CHUNK)], tbuf)

        def inner(i, _):
            sample16(i, cbuf, lbuf, tbuf)
            return 0

        return lax.fori_loop(0, IN_IT, inner, 0)

    lax.fori_loop(0, KCH, chunk_body, 0)

    @pl.when(wid == 0)
    def _tail():
        pltpu.sync_copy(conf_hbm.at[pl.ds(MAIN, TAIL)], ebuf_c)
        pltpu.sync_copy(logf_hbm.at[pl.ds(MAIN * 3, TAIL * 3)], ebuf_l)
        pltpu.sync_copy(tgt_hbm.at[pl.ds(MAIN, TAIL)], ebuf_t)
        for i in range(TAIL // 16):
            sample16(i, ebuf_c, ebuf_l, ebuf_t)

    for g in range(4):
        vc = zero16
        vs = zero16
        vr = zero16
        for l in range(16):
            o = l * NBINS + g * 16
            vc = vc + hc[pl.ds(o, 16)]
            vs = vs + hs[pl.ds(o, 16)]
            vr = vr + hr[pl.ds(o, 16)]
        prow[pl.ds(g * 16, 16)] = vc
        prow[pl.ds(64 + g * 16, 16)] = vs
        prow[pl.ds(128 + g * 16, 16)] = vr
    prow[pl.ds(192, 16)] = acc[...]
    prow[pl.ds(208, 16)] = zero16
    prow[pl.ds(224, 16)] = zero16
    prow[pl.ds(240, 16)] = zero16
    pltpu.sync_copy(prow, out_hbm.at[pl.ds(wid * ROW, ROW)])


_sc_hist = functools.partial(
    pl.kernel,
    out_type=jax.ShapeDtypeStruct((NW * ROW,), jnp.float32),
    mesh=plsc.VectorSubcoreMesh(core_axis_name="c", subcore_axis_name="s"),
    compiler_params=pltpu.CompilerParams(needs_layout_passes=False),
    scratch_types=[
        pltpu.VMEM((CHUNK,), jnp.float32),
        pltpu.VMEM((CHUNK * 3,), jnp.float32),
        pltpu.VMEM((CHUNK,), jnp.int32),
        pltpu.VMEM((TAIL,), jnp.float32),
        pltpu.VMEM((TAIL * 3,), jnp.float32),
        pltpu.VMEM((TAIL,), jnp.int32),
        pltpu.VMEM((56,), jnp.float32),
        pltpu.VMEM((16 * NBINS,), jnp.float32),
        pltpu.VMEM((16 * NBINS,), jnp.float32),
        pltpu.VMEM((16 * NBINS,), jnp.float32),
        pltpu.VMEM((16,), jnp.float32),
        pltpu.VMEM((ROW,), jnp.float32),
    ],
)(_sc_body)


def _combine(x_ref, t_ref, b_ref, e_ref):
    x = x_ref[...]
    nf = jnp.float32(N)
    cnt = jnp.sum(x[:, 0:64], axis=0, keepdims=True)
    sconf = jnp.sum(x[:, 64:128], axis=0, keepdims=True)
    scorr = jnp.sum(x[:, 128:192], axis=0, keepdims=True)
    bce = jnp.sum(x[:, 192:208]) / nf
    safe = jnp.maximum(cnt, 1.0)
    term = jnp.where(cnt > 0, (cnt / nf) * jnp.abs(scorr / safe - sconf / safe), 0.0)
    ece = jnp.sum(term)
    t_ref[0, 0] = bce + ece
    b_ref[0, 0] = bce
    e_ref[0, 0] = ece


def kernel(confidence, direction_logits, targets):
    conf = confidence.reshape(N)
    logf = direction_logits.reshape(N * 3)
    bounds = jnp.pad(jnp.linspace(0.0, 1.0, 51), (0, 5), constant_values=2.0)
    partial = _sc_hist(conf, logf, targets, bounds)
    total, bce, ece = pl.pallas_call(
        _combine,
        out_shape=(
            jax.ShapeDtypeStruct((1, 1), jnp.float32),
            jax.ShapeDtypeStruct((1, 1), jnp.float32),
            jax.ShapeDtypeStruct((1, 1), jnp.float32),
        ),
        out_specs=(
            pl.BlockSpec(memory_space=pltpu.SMEM),
            pl.BlockSpec(memory_space=pltpu.SMEM),
            pl.BlockSpec(memory_space=pltpu.SMEM),
        ),
    )(partial.reshape(NW, ROW))
    return (total[0, 0], bce[0, 0], ece[0, 0])


# speedup vs baseline: 14.1684x; 14.1684x over previous
import functools

import jax
import jax.numpy as jnp
from jax import lax
from jax.experimental import pallas as pl
from jax.experimental.pallas import tpu as pltpu
from jax.experimental.pallas import tpu_sc as plsc

N = 1_000_000
NW = 32
PER_W = 31_248
MAIN = NW * PER_W
TAIL = N - MAIN
KCH = 7
CHUNK = PER_W // KCH
IN_IT = CHUNK // 16
NBINS = 64
ROW = 256
LN2 = 0.69314718055994530942
SQRT2 = 1.4142135623730951


def _vlog(x):
    bits = plsc.bitcast(x, jnp.int32)
    e = (bits >> 23) - 127
    m = plsc.bitcast((bits & 0x007FFFFF) | 0x3F800000, jnp.float32)
    big = m > SQRT2
    m = jnp.where(big, m * 0.5, m)
    ef = (e + big.astype(jnp.int32)).astype(jnp.float32)
    s = (m - 1.0) / (m + 1.0)
    t = s * s
    poly = 1.0 + t * (1.0 / 3.0 + t * (1.0 / 5.0 + t * (1.0 / 7.0 + t * (1.0 / 9.0))))
    return ef * LN2 + 2.0 * s * poly


def _sc_body(conf_hbm, l0_hbm, l1_hbm, l2_hbm, tgt_hbm, bnd_hbm, out_hbm,
             cbuf, b0, b1, b2, tbuf, ec, e0, e1, e2, et, bnd, hc, hs, hr, acc, prow):
    nc = 2
    wid = lax.axis_index("s") * nc + lax.axis_index("c")
    base = wid * PER_W
    lane = lax.iota(jnp.int32, 16)
    zero16 = jnp.zeros((16,), jnp.float32)

    pltpu.sync_copy(bnd_hbm, bnd)
    for i in range(NBINS):
        hc[pl.ds(i * 16, 16)] = zero16
        hs[pl.ds(i * 16, 16)] = zero16
        hr[pl.ds(i * 16, 16)] = zero16
    acc[...] = zero16

    def sample16(i, cb, lb0, lb1, lb2, tb):
        conf = cb[pl.ds(i * 16, 16)]
        tgt = tb[pl.ds(i * 16, 16)]
        l0 = lb0[pl.ds(i * 16, 16)]
        l1 = lb1[pl.ds(i * 16, 16)]
        l2 = lb2[pl.ds(i * 16, 16)]
        pred = jnp.where(l2 > jnp.maximum(l0, l1), 2, jnp.where(l1 > l0, 1, 0))
        corr = (pred == tgt).astype(jnp.float32)
        p = jnp.maximum(conf, 1e-12)
        q = jnp.where(corr > 0.5, p, 1.0 - p)
        acc[...] = acc[...] - _vlog(q)
        y = conf * 50.0
        iy = y.astype(jnp.int32)
        j = iy + (y > iy.astype(jnp.float32)).astype(jnp.int32) - 1
        jc = jnp.clip(j, 0, 49)
        lo = plsc.load_gather(bnd, [jc])
        hi = plsc.load_gather(bnd, [jc + 1])
        j2 = jc + (conf > hi).astype(jnp.int32) - (conf <= lo).astype(jnp.int32)
        valid = j2 >= 0
        hidx = lane * NBINS + jnp.maximum(j2, 0)
        plsc.addupdate_scatter(hc, [hidx], jnp.ones((16,), jnp.float32), mask=valid)
        plsc.addupdate_scatter(hs, [hidx], conf, mask=valid)
        plsc.addupdate_scatter(hr, [hidx], corr, mask=valid)

    def chunk_body(g, _):
        off = base + g * CHUNK
        pltpu.sync_copy(conf_hbm.at[pl.ds(off, CHUNK)], cbuf)
        pltpu.sync_copy(l0_hbm.at[pl.ds(off, CHUNK)], b0)
        pltpu.sync_copy(l1_hbm.at[pl.ds(off, CHUNK)], b1)
        pltpu.sync_copy(l2_hbm.at[pl.ds(off, CHUNK)], b2)
        pltpu.sync_copy(tgt_hbm.at[pl.ds(off, CHUNK)], tbuf)

        def inner(i, _):
            sample16(i, cbuf, b0, b1, b2, tbuf)
            return 0

        return lax.fori_loop(0, IN_IT, inner, 0)

    lax.fori_loop(0, KCH, chunk_body, 0)

    @pl.when(wid == 0)
    def _tail():
        pltpu.sync_copy(conf_hbm.at[pl.ds(MAIN, TAIL)], ec)
        pltpu.sync_copy(l0_hbm.at[pl.ds(MAIN, TAIL)], e0)
        pltpu.sync_copy(l1_hbm.at[pl.ds(MAIN, TAIL)], e1)
        pltpu.sync_copy(l2_hbm.at[pl.ds(MAIN, TAIL)], e2)
        pltpu.sync_copy(tgt_hbm.at[pl.ds(MAIN, TAIL)], et)
        for i in range(TAIL // 16):
            sample16(i, ec, e0, e1, e2, et)

    for g in range(4):
        vc = zero16
        vs = zero16
        vr = zero16
        for l in range(16):
            o = l * NBINS + g * 16
            vc = vc + hc[pl.ds(o, 16)]
            vs = vs + hs[pl.ds(o, 16)]
            vr = vr + hr[pl.ds(o, 16)]
        prow[pl.ds(g * 16, 16)] = vc
        prow[pl.ds(64 + g * 16, 16)] = vs
        prow[pl.ds(128 + g * 16, 16)] = vr
    prow[pl.ds(192, 16)] = acc[...]
    prow[pl.ds(208, 16)] = zero16
    prow[pl.ds(224, 16)] = zero16
    prow[pl.ds(240, 16)] = zero16
    pltpu.sync_copy(prow, out_hbm.at[pl.ds(wid * ROW, ROW)])


_sc_hist = functools.partial(
    pl.kernel,
    out_type=jax.ShapeDtypeStruct((NW * ROW,), jnp.float32),
    mesh=plsc.VectorSubcoreMesh(core_axis_name="c", subcore_axis_name="s"),
    compiler_params=pltpu.CompilerParams(needs_layout_passes=False),
    scratch_types=[
        pltpu.VMEM((CHUNK,), jnp.float32),
        pltpu.VMEM((CHUNK,), jnp.float32),
        pltpu.VMEM((CHUNK,), jnp.float32),
        pltpu.VMEM((CHUNK,), jnp.float32),
        pltpu.VMEM((CHUNK,), jnp.int32),
        pltpu.VMEM((TAIL,), jnp.float32),
        pltpu.VMEM((TAIL,), jnp.float32),
        pltpu.VMEM((TAIL,), jnp.float32),
        pltpu.VMEM((TAIL,), jnp.float32),
        pltpu.VMEM((TAIL,), jnp.int32),
        pltpu.VMEM((56,), jnp.float32),
        pltpu.VMEM((16 * NBINS,), jnp.float32),
        pltpu.VMEM((16 * NBINS,), jnp.float32),
        pltpu.VMEM((16 * NBINS,), jnp.float32),
        pltpu.VMEM((16,), jnp.float32),
        pltpu.VMEM((ROW,), jnp.float32),
    ],
)(_sc_body)


def _combine(x_ref, t_ref, b_ref, e_ref):
    x = x_ref[...]
    nf = jnp.float32(N)
    cnt = jnp.sum(x[:, 0:64], axis=0, keepdims=True)
    sconf = jnp.sum(x[:, 64:128], axis=0, keepdims=True)
    scorr = jnp.sum(x[:, 128:192], axis=0, keepdims=True)
    bce = jnp.sum(x[:, 192:208]) / nf
    safe = jnp.maximum(cnt, 1.0)
    term = jnp.where(cnt > 0, (cnt / nf) * jnp.abs(scorr / safe - sconf / safe), 0.0)
    ece = jnp.sum(term)
    t_ref[0, 0] = bce + ece
    b_ref[0, 0] = bce
    e_ref[0, 0] = ece


def kernel(confidence, direction_logits, targets):
    conf = jnp.squeeze(confidence, -1)
    l0 = direction_logits[:, 0]
    l1 = direction_logits[:, 1]
    l2 = direction_logits[:, 2]
    bounds = jnp.pad(jnp.linspace(0.0, 1.0, 51), (0, 5), constant_values=2.0)
    partial = _sc_hist(conf, l0, l1, l2, targets, bounds)
    total, bce, ece = pl.pallas_call(
        _combine,
        out_shape=(
            jax.ShapeDtypeStruct((1, 1), jnp.float32),
            jax.ShapeDtypeStruct((1, 1), jnp.float32),
            jax.ShapeDtypeStruct((1, 1), jnp.float32),
        ),
        out_specs=(
            pl.BlockSpec(memory_space=pltpu.SMEM),
            pl.BlockSpec(memory_space=pltpu.SMEM),
            pl.BlockSpec(memory_space=pltpu.SMEM),
        ),
    )(partial.reshape(NW, ROW))
    return (total[0, 0], bce[0, 0], ece[0, 0])


# speedup vs baseline: 19.2809x; 1.3608x over previous
import functools

import jax
import jax.numpy as jnp
from jax import lax
from jax.experimental import pallas as pl
from jax.experimental.pallas import tpu as pltpu
from jax.experimental.pallas import tpu_sc as plsc

N = 1_000_000
NW = 32
PER_W = 31_232
MAIN = NW * PER_W
TAIL = N - MAIN
KCH = 4
CHUNK = PER_W // KCH
IN_IT = CHUNK // 16
NBINS = 64
ROW = 256
LN2 = 0.69314718055994530942
SQRT2 = 1.4142135623730951


def _vlog(x):
    bits = plsc.bitcast(x, jnp.int32)
    e = (bits >> 23) - 127
    m = plsc.bitcast((bits & 0x007FFFFF) | 0x3F800000, jnp.float32)
    big = m > SQRT2
    m = jnp.where(big, m * 0.5, m)
    ef = (e + big.astype(jnp.int32)).astype(jnp.float32)
    s = (m - 1.0) / (m + 1.0)
    t = s * s
    poly = 1.0 + t * (1.0 / 3.0 + t * (1.0 / 5.0 + t * (1.0 / 7.0 + t * (1.0 / 9.0))))
    return ef * LN2 + 2.0 * s * poly


def _sc_body(conf_hbm, lf_hbm, tgt_hbm, bnd_hbm, out_hbm,
             cbuf, b0, b1, b2, tbuf, ec, e0, e1, e2, et, bnd, hc, hs, hr, acc, prow):
    nc = 2
    wid = lax.axis_index("s") * nc + lax.axis_index("c")
    base = wid * PER_W
    lane = lax.iota(jnp.int32, 16)
    zero16 = jnp.zeros((16,), jnp.float32)

    pltpu.sync_copy(bnd_hbm, bnd)
    for i in range(NBINS):
        hc[pl.ds(i * 16, 16)] = zero16
        hs[pl.ds(i * 16, 16)] = zero16
        hr[pl.ds(i * 16, 16)] = zero16
    acc[...] = zero16

    def sample16(i, cb, lb0, lb1, lb2, tb):
        conf = cb[pl.ds(i * 16, 16)]
        tgt = tb[pl.ds(i * 16, 16)]
        l0 = lb0[pl.ds(i * 16, 16)]
        l1 = lb1[pl.ds(i * 16, 16)]
        l2 = lb2[pl.ds(i * 16, 16)]
        pred = jnp.where(l2 > jnp.maximum(l0, l1), 2, jnp.where(l1 > l0, 1, 0))
        corr = (pred == tgt).astype(jnp.float32)
        p = jnp.maximum(conf, 1e-12)
        q = jnp.where(corr > 0.5, p, 1.0 - p)
        acc[...] = acc[...] - _vlog(q)
        y = conf * 50.0
        iy = y.astype(jnp.int32)
        j = iy + (y > iy.astype(jnp.float32)).astype(jnp.int32) - 1
        jc = jnp.clip(j, 0, 49)
        lo = plsc.load_gather(bnd, [jc])
        hi = plsc.load_gather(bnd, [jc + 1])
        j2 = jc + (conf > hi).astype(jnp.int32) - (conf <= lo).astype(jnp.int32)
        valid = j2 >= 0
        hidx = lane * NBINS + jnp.maximum(j2, 0)
        plsc.addupdate_scatter(hc, [hidx], jnp.ones((16,), jnp.float32), mask=valid)
        plsc.addupdate_scatter(hs, [hidx], conf, mask=valid)
        plsc.addupdate_scatter(hr, [hidx], corr, mask=valid)

    def chunk_body(g, _):
        off = base + g * CHUNK
        pltpu.sync_copy(conf_hbm.at[0, pl.ds(off, CHUNK)], cbuf)
        pltpu.sync_copy(lf_hbm.at[pl.ds(off, CHUNK)], b0)
        pltpu.sync_copy(lf_hbm.at[pl.ds(N + off, CHUNK)], b1)
        pltpu.sync_copy(lf_hbm.at[pl.ds(2 * N + off, CHUNK)], b2)
        pltpu.sync_copy(tgt_hbm.at[pl.ds(off, CHUNK)], tbuf)

        def inner(i, _):
            sample16(i, cbuf, b0, b1, b2, tbuf)
            return 0

        return lax.fori_loop(0, IN_IT, inner, 0)

    lax.fori_loop(0, KCH, chunk_body, 0)

    @pl.when(wid == 0)
    def _tail():
        pltpu.sync_copy(conf_hbm.at[0, pl.ds(MAIN, TAIL)], ec)
        pltpu.sync_copy(lf_hbm.at[pl.ds(MAIN, TAIL)], e0)
        pltpu.sync_copy(lf_hbm.at[pl.ds(N + MAIN, TAIL)], e1)
        pltpu.sync_copy(lf_hbm.at[pl.ds(2 * N + MAIN, TAIL)], e2)
        pltpu.sync_copy(tgt_hbm.at[pl.ds(MAIN, TAIL)], et)
        for i in range(TAIL // 16):
            sample16(i, ec, e0, e1, e2, et)

    for g in range(4):
        vc = zero16
        vs = zero16
        vr = zero16
        for l in range(16):
            o = l * NBINS + g * 16
            vc = vc + hc[pl.ds(o, 16)]
            vs = vs + hs[pl.ds(o, 16)]
            vr = vr + hr[pl.ds(o, 16)]
        prow[pl.ds(g * 16, 16)] = vc
        prow[pl.ds(64 + g * 16, 16)] = vs
        prow[pl.ds(128 + g * 16, 16)] = vr
    prow[pl.ds(192, 16)] = acc[...]
    prow[pl.ds(208, 16)] = zero16
    prow[pl.ds(224, 16)] = zero16
    prow[pl.ds(240, 16)] = zero16
    pltpu.sync_copy(prow, out_hbm.at[pl.ds(wid * ROW, ROW)])


_sc_hist = functools.partial(
    pl.kernel,
    out_type=jax.ShapeDtypeStruct((NW * ROW,), jnp.float32),
    mesh=plsc.VectorSubcoreMesh(core_axis_name="c", subcore_axis_name="s"),
    compiler_params=pltpu.CompilerParams(needs_layout_passes=False),
    scratch_types=[
        pltpu.VMEM((CHUNK,), jnp.float32),
        pltpu.VMEM((CHUNK,), jnp.float32),
        pltpu.VMEM((CHUNK,), jnp.float32),
        pltpu.VMEM((CHUNK,), jnp.float32),
        pltpu.VMEM((CHUNK,), jnp.int32),
        pltpu.VMEM((TAIL,), jnp.float32),
        pltpu.VMEM((TAIL,), jnp.float32),
        pltpu.VMEM((TAIL,), jnp.float32),
        pltpu.VMEM((TAIL,), jnp.float32),
        pltpu.VMEM((TAIL,), jnp.int32),
        pltpu.VMEM((56,), jnp.float32),
        pltpu.VMEM((16 * NBINS,), jnp.float32),
        pltpu.VMEM((16 * NBINS,), jnp.float32),
        pltpu.VMEM((16 * NBINS,), jnp.float32),
        pltpu.VMEM((16,), jnp.float32),
        pltpu.VMEM((ROW,), jnp.float32),
    ],
)(_sc_body)


def _combine(x_ref, t_ref, b_ref, e_ref):
    x = x_ref[...]
    nf = jnp.float32(N)
    cnt = jnp.sum(x[:, 0:64], axis=0, keepdims=True)
    sconf = jnp.sum(x[:, 64:128], axis=0, keepdims=True)
    scorr = jnp.sum(x[:, 128:192], axis=0, keepdims=True)
    bce = jnp.sum(x[:, 192:208]) / nf
    safe = jnp.maximum(cnt, 1.0)
    term = jnp.where(cnt > 0, (cnt / nf) * jnp.abs(scorr / safe - sconf / safe), 0.0)
    ece = jnp.sum(term)
    t_ref[0, 0] = bce + ece
    b_ref[0, 0] = bce
    e_ref[0, 0] = ece


def kernel(confidence, direction_logits, targets):
    conf = confidence.T
    lflat = direction_logits.T.reshape(3 * N)
    bounds = jnp.pad(jnp.linspace(0.0, 1.0, 51), (0, 5), constant_values=2.0)
    partial = _sc_hist(conf, lflat, targets, bounds)
    total, bce, ece = pl.pallas_call(
        _combine,
        out_shape=(
            jax.ShapeDtypeStruct((1, 1), jnp.float32),
            jax.ShapeDtypeStruct((1, 1), jnp.float32),
            jax.ShapeDtypeStruct((1, 1), jnp.float32),
        ),
        out_specs=(
            pl.BlockSpec(memory_space=pltpu.SMEM),
            pl.BlockSpec(memory_space=pltpu.SMEM),
            pl.BlockSpec(memory_space=pltpu.SMEM),
        ),
    )(partial.reshape(NW, ROW))
    return (total[0, 0], bce[0, 0], ece[0, 0])


# speedup vs baseline: 24.3327x; 1.2620x over previous
import functools

import jax
import jax.numpy as jnp
from jax import lax
from jax.experimental import pallas as pl
from jax.experimental.pallas import tpu as pltpu
from jax.experimental.pallas import tpu_sc as plsc

N = 1_000_000
NW = 32
PER_W = 31_232
MAIN = NW * PER_W
TAIL = N - MAIN
KCH = 4
CHUNK = PER_W // KCH
UNROLL = 4
IN_IT = CHUNK // 16
NBINS = 64
ROW = 256
LN2 = 0.69314718055994530942
SQRT2 = 1.4142135623730951


def _vlog(x):
    bits = plsc.bitcast(x, jnp.int32)
    e = (bits >> 23) - 127
    m = plsc.bitcast((bits & 0x007FFFFF) | 0x3F800000, jnp.float32)
    big = m > SQRT2
    m = jnp.where(big, m * 0.5, m)
    ef = (e + big.astype(jnp.int32)).astype(jnp.float32)
    s = (m - 1.0) / (m + 1.0)
    t = s * s
    poly = 1.0 + t * (1.0 / 3.0 + t * (1.0 / 5.0 + t * (1.0 / 7.0 + t * (1.0 / 9.0))))
    return ef * LN2 + 2.0 * s * poly


def _sc_body(conf_hbm, lf_hbm, tgt_hbm, bnd_hbm, out_hbm,
             cbuf, b0, b1, b2, tbuf, ec, e0, e1, e2, et, bnd, hc, hs, hr, acc, prow):
    nc = 2
    wid = lax.axis_index("s") * nc + lax.axis_index("c")
    base = wid * PER_W
    lane = lax.iota(jnp.int32, 16)
    zero16 = jnp.zeros((16,), jnp.float32)

    pltpu.sync_copy(bnd_hbm, bnd)
    for i in range(NBINS):
        hc[pl.ds(i * 16, 16)] = zero16
        hs[pl.ds(i * 16, 16)] = zero16
        hr[pl.ds(i * 16, 16)] = zero16

    def sample16(i, cb, lb0, lb1, lb2, tb):
        conf = cb[pl.ds(i * 16, 16)]
        tgt = tb[pl.ds(i * 16, 16)]
        l0 = lb0[pl.ds(i * 16, 16)]
        l1 = lb1[pl.ds(i * 16, 16)]
        l2 = lb2[pl.ds(i * 16, 16)]
        pred = jnp.where(l2 > jnp.maximum(l0, l1), 2, jnp.where(l1 > l0, 1, 0))
        corr = (pred == tgt).astype(jnp.float32)
        p = jnp.maximum(conf, 1e-12)
        q = jnp.where(corr > 0.5, p, 1.0 - p)
        y = conf * 50.0
        iy = y.astype(jnp.int32)
        j = iy + (y > iy.astype(jnp.float32)).astype(jnp.int32) - 1
        jc = jnp.clip(j, 0, 49)
        lo = plsc.load_gather(bnd, [jc])
        hi = plsc.load_gather(bnd, [jc + 1])
        j2 = jc + (conf > hi).astype(jnp.int32) - (conf <= lo).astype(jnp.int32)
        valid = j2 >= 0
        hidx = lane * NBINS + jnp.maximum(j2, 0)
        plsc.addupdate_scatter(hc, [hidx], jnp.ones((16,), jnp.float32), mask=valid)
        plsc.addupdate_scatter(hs, [hidx], conf, mask=valid)
        plsc.addupdate_scatter(hr, [hidx], corr, mask=valid)
        return -_vlog(q)

    def chunk_body(g, a):
        off = base + g * CHUNK
        pltpu.sync_copy(conf_hbm.at[0, pl.ds(off, CHUNK)], cbuf)
        pltpu.sync_copy(lf_hbm.at[pl.ds(off, CHUNK)], b0)
        pltpu.sync_copy(lf_hbm.at[pl.ds(N + off, CHUNK)], b1)
        pltpu.sync_copy(lf_hbm.at[pl.ds(2 * N + off, CHUNK)], b2)
        pltpu.sync_copy(tgt_hbm.at[pl.ds(off, CHUNK)], tbuf)

        def inner(i, a2):
            for u in range(UNROLL):
                a2 = a2 + sample16(i * UNROLL + u, cbuf, b0, b1, b2, tbuf)
            return a2

        return lax.fori_loop(0, IN_IT // UNROLL, inner, a)

    acc_main = lax.fori_loop(0, KCH, chunk_body, zero16)
    acc[...] = zero16

    @pl.when(wid == 0)
    def _tail():
        pltpu.sync_copy(conf_hbm.at[0, pl.ds(MAIN, TAIL)], ec)
        pltpu.sync_copy(lf_hbm.at[pl.ds(MAIN, TAIL)], e0)
        pltpu.sync_copy(lf_hbm.at[pl.ds(N + MAIN, TAIL)], e1)
        pltpu.sync_copy(lf_hbm.at[pl.ds(2 * N + MAIN, TAIL)], e2)
        pltpu.sync_copy(tgt_hbm.at[pl.ds(MAIN, TAIL)], et)
        a2 = zero16
        for i in range(TAIL // 16):
            a2 = a2 + sample16(i, ec, e0, e1, e2, et)
        acc[...] = a2

    for g in range(4):
        vc = zero16
        vs = zero16
        vr = zero16
        for l in range(16):
            o = l * NBINS + g * 16
            vc = vc + hc[pl.ds(o, 16)]
            vs = vs + hs[pl.ds(o, 16)]
            vr = vr + hr[pl.ds(o, 16)]
        prow[pl.ds(g * 16, 16)] = vc
        prow[pl.ds(64 + g * 16, 16)] = vs
        prow[pl.ds(128 + g * 16, 16)] = vr
    prow[pl.ds(192, 16)] = acc_main + acc[...]
    prow[pl.ds(208, 16)] = zero16
    prow[pl.ds(224, 16)] = zero16
    prow[pl.ds(240, 16)] = zero16
    pltpu.sync_copy(prow, out_hbm.at[pl.ds(wid * ROW, ROW)])


_sc_hist = functools.partial(
    pl.kernel,
    out_type=jax.ShapeDtypeStruct((NW * ROW,), jnp.float32),
    mesh=plsc.VectorSubcoreMesh(core_axis_name="c", subcore_axis_name="s"),
    compiler_params=pltpu.CompilerParams(needs_layout_passes=False),
    scratch_types=[
        pltpu.VMEM((CHUNK,), jnp.float32),
        pltpu.VMEM((CHUNK,), jnp.float32),
        pltpu.VMEM((CHUNK,), jnp.float32),
        pltpu.VMEM((CHUNK,), jnp.float32),
        pltpu.VMEM((CHUNK,), jnp.int32),
        pltpu.VMEM((TAIL,), jnp.float32),
        pltpu.VMEM((TAIL,), jnp.float32),
        pltpu.VMEM((TAIL,), jnp.float32),
        pltpu.VMEM((TAIL,), jnp.float32),
        pltpu.VMEM((TAIL,), jnp.int32),
        pltpu.VMEM((56,), jnp.float32),
        pltpu.VMEM((16 * NBINS,), jnp.float32),
        pltpu.VMEM((16 * NBINS,), jnp.float32),
        pltpu.VMEM((16 * NBINS,), jnp.float32),
        pltpu.VMEM((16,), jnp.float32),
        pltpu.VMEM((ROW,), jnp.float32),
    ],
)(_sc_body)


def _combine(x_ref, t_ref, b_ref, e_ref):
    x = x_ref[...]
    nf = jnp.float32(N)
    cnt = jnp.sum(x[:, 0:64], axis=0, keepdims=True)
    sconf = jnp.sum(x[:, 64:128], axis=0, keepdims=True)
    scorr = jnp.sum(x[:, 128:192], axis=0, keepdims=True)
    bce = jnp.sum(x[:, 192:208]) / nf
    safe = jnp.maximum(cnt, 1.0)
    term = jnp.where(cnt > 0, (cnt / nf) * jnp.abs(scorr / safe - sconf / safe), 0.0)
    ece = jnp.sum(term)
    t_ref[0, 0] = bce + ece
    b_ref[0, 0] = bce
    e_ref[0, 0] = ece


def kernel(confidence, direction_logits, targets):
    conf = confidence.T
    lflat = direction_logits.T.reshape(3 * N)
    bounds = jnp.pad(jnp.linspace(0.0, 1.0, 51), (0, 5), constant_values=2.0)
    partial = _sc_hist(conf, lflat, targets, bounds)
    total, bce, ece = pl.pallas_call(
        _combine,
        out_shape=(
            jax.ShapeDtypeStruct((1, 1), jnp.float32),
            jax.ShapeDtypeStruct((1, 1), jnp.float32),
            jax.ShapeDtypeStruct((1, 1), jnp.float32),
        ),
        out_specs=(
            pl.BlockSpec(memory_space=pltpu.SMEM),
            pl.BlockSpec(memory_space=pltpu.SMEM),
            pl.BlockSpec(memory_space=pltpu.SMEM),
        ),
    )(partial.reshape(NW, ROW))
    return (total[0, 0], bce[0, 0], ece[0, 0])
